# baseline (speedup 1.0000x reference)
import jax
import jax.numpy as jnp
from jax.experimental import pallas as pl
from jax.experimental.pallas import tpu as pltpu

_N, _H, _W, _C = 32, 56, 56, 256
_OH = _OW = 29
_CB = 128
_INV9 = 1.0 / 9.0
_NDCOL = 9.0 * _N * _C * _OH * _OW
_PW = 66


def _dsum_kernel(x_ref, out_ref, img_s):
    img_s[0:2] = jnp.zeros((2, _PW, _C), jnp.float32)
    img_s[58:60] = jnp.zeros((2, _PW, _C), jnp.float32)
    img_s[2:58, 0:8] = jnp.zeros((56, 8, _C), jnp.float32)
    img_s[2:58, 64:66] = jnp.zeros((56, 2, _C), jnp.float32)
    img_s[2:58, 8:64] = x_ref[0]
    c2 = img_s[:, 8:66]
    c2r = c2[2:60]
    c0r = jnp.concatenate(
        [jnp.zeros((58, 2, _C), jnp.float32), c2r[:, 0:56]], axis=1)
    d = jnp.abs(c2r - c2[0:58]) + jnp.abs(c2r - c0r)
    dv = d.reshape(29, 2, 58, _C)
    dev = dv[:, 0]
    dod = dv[:, 1]
    hsum = (2.0 * jnp.sum(dev[0:28], axis=0)
            + jnp.sum(dod[0:28], axis=0)
            + dev[28])
    j = jax.lax.broadcasted_iota(jnp.int32, (58, 1), 0)
    w = jnp.where(j == 57, 0.0,
                  jnp.where(j == 56, 1.0,
                            jnp.where(j % 2 == 0, 2.0, 1.0)))
    out_ref[0, 0] = jnp.sum(hsum * w, axis=0, keepdims=True)


def _pool_kernel(part_ref, x_ref, out_ref, rm_s, rs_s, cen_s):
    t = jnp.sum(part_ref[...].reshape(_N, _C)) / _NDCOL
    for h in range(_C // _CB):
        sl = slice(h * _CB, (h + 1) * _CB)
        img = jnp.pad(x_ref[0, :, :, sl], ((2, 2), (2, 2), (0, 0)))

        iv = img.reshape(30, 2, 60, _CB)
        ev, od = iv[:, 0], iv[:, 1]
        lA, lB, lC = ev[0:29], od[0:29], ev[1:30]
        rm_s[...] = jnp.maximum(jnp.maximum(lA, lB), lC)
        rs_s[...] = lA + lB + lC

        mA = rm_s[:, pl.ds(0, 29, 2), :]
        mB = rm_s[:, pl.ds(1, 29, 2), :]
        mC = rm_s[:, pl.ds(2, 29, 2), :]
        pmax = jnp.maximum(jnp.maximum(mA, mB), mC)
        sA = rs_s[:, pl.ds(0, 29, 2), :]
        sB = rs_s[:, pl.ds(1, 29, 2), :]
        sC = rs_s[:, pl.ds(2, 29, 2), :]
        pmean = (sA + sB + sC) * _INV9

        aodd = od[1:29] - od[0:28]
        bodd = od[1:29]
        dodd = (jnp.abs(aodd[:, 2:60])
                + jnp.abs(bodd[:, 2:60] - bodd[:, 0:58]))
        cen_s[...] = dodd
        cc = cen_s[:, pl.ds(1, 28, 2), :]
        center = jnp.pad(cc, ((1, 0), (1, 0), (0, 0)))

        out_ref[0, :, :, sl] = jnp.where(center > t, pmax, pmean)


def kernel(x):
    partial = pl.pallas_call(
        _dsum_kernel,
        grid=(_N,),
        in_specs=[pl.BlockSpec((1, _H, _W, _C), lambda i: (i, 0, 0, 0))],
        out_specs=pl.BlockSpec((1, 1, 1, _C), lambda i: (i, 0, 0, 0)),
        out_shape=jax.ShapeDtypeStruct((_N, 1, 1, _C), jnp.float32),
        scratch_shapes=[pltpu.VMEM((60, _PW, _C), jnp.float32)],
        compiler_params=pltpu.CompilerParams(
            dimension_semantics=("parallel",)),
    )(x)
    return pl.pallas_call(
        _pool_kernel,
        grid=(_N,),
        in_specs=[
            pl.BlockSpec((_N, 1, 1, _C), lambda i: (0, 0, 0, 0)),
            pl.BlockSpec((1, _H, _W, _C), lambda i: (i, 0, 0, 0)),
        ],
        out_specs=pl.BlockSpec((1, _OH, _OW, _C), lambda i: (i, 0, 0, 0)),
        out_shape=jax.ShapeDtypeStruct((_N, _OH, _OW, _C), jnp.float32),
        scratch_shapes=[
            pltpu.VMEM((_OH, 60, _CB), jnp.float32),
            pltpu.VMEM((_OH, 60, _CB), jnp.float32),
            pltpu.VMEM((28, 58, _CB), jnp.float32),
        ],
        compiler_params=pltpu.CompilerParams(
            dimension_semantics=("parallel",)),
    )(partial, x)


# speedup vs baseline: 1.1192x; 1.1192x over previous
import jax
import jax.numpy as jnp
from jax.experimental import pallas as pl
from jax.experimental.pallas import tpu as pltpu

_N, _H, _W, _C = 32, 56, 56, 256
_OH = _OW = 29
_CB = 128
_INV9 = 1.0 / 9.0
_NDCOL = 9.0 * _N * _C * _OH * _OW
_PW = 66


def _fused_kernel(x_ref, out_ref, img_s, rm_s, rs_s, cen_s, acc_s, t_s):
    i = pl.program_id(0)

    @pl.when(i < _N)
    def _pass1():
        img_s[0:2] = jnp.zeros((2, _PW, _C), jnp.float32)
        img_s[58:60] = jnp.zeros((2, _PW, _C), jnp.float32)
        img_s[2:58, 0:8] = jnp.zeros((56, 8, _C), jnp.float32)
        img_s[2:58, 64:66] = jnp.zeros((56, 2, _C), jnp.float32)
        img_s[2:58, 8:64] = x_ref[0]
        c2 = img_s[:, 8:66]
        c2r = c2[2:60]
        c0r = jnp.concatenate(
            [jnp.zeros((58, 2, _C), jnp.float32), c2r[:, 0:56]], axis=1)
        d = jnp.abs(c2r - c2[0:58]) + jnp.abs(c2r - c0r)
        dv = d.reshape(29, 2, 58, _C)
        dev = dv[:, 0]
        dod = dv[:, 1]
        hsum = (2.0 * jnp.sum(dev[0:28], axis=0)
                + jnp.sum(dod[0:28], axis=0)
                + dev[28])
        j = jax.lax.broadcasted_iota(jnp.int32, (58, 1), 0)
        w = jnp.where(j == 57, 0.0,
                      jnp.where(j == 56, 1.0,
                                jnp.where(j % 2 == 0, 2.0, 1.0)))
        pv = jnp.sum(hsum * w, axis=0, keepdims=True)
        acc_s[...] = jnp.where(i == 0, pv, acc_s[...] + pv)

    @pl.when(i >= _N)
    def _pass2():
        @pl.when(i == _N)
        def _compute_t():
            t_s[0] = jnp.sum(acc_s[...]) / _NDCOL

        t = t_s[0]
        for h in range(_C // _CB):
            sl = slice(h * _CB, (h + 1) * _CB)
            img = jnp.pad(x_ref[0, :, :, sl], ((2, 2), (2, 2), (0, 0)))

            iv = img.reshape(30, 2, 60, _CB)
            ev, od = iv[:, 0], iv[:, 1]
            lA, lB, lC = ev[0:29], od[0:29], ev[1:30]
            rm_s[...] = jnp.maximum(jnp.maximum(lA, lB), lC)
            rs_s[...] = lA + lB + lC

            mA = rm_s[:, pl.ds(0, 29, 2), :]
            mB = rm_s[:, pl.ds(1, 29, 2), :]
            mC = rm_s[:, pl.ds(2, 29, 2), :]
            pmax = jnp.maximum(jnp.maximum(mA, mB), mC)
            sA = rs_s[:, pl.ds(0, 29, 2), :]
            sB = rs_s[:, pl.ds(1, 29, 2), :]
            sC = rs_s[:, pl.ds(2, 29, 2), :]
            pmean = (sA + sB + sC) * _INV9

            aodd = od[1:29] - od[0:28]
            bodd = od[1:29]
            dodd = (jnp.abs(aodd[:, 2:60])
                    + jnp.abs(bodd[:, 2:60] - bodd[:, 0:58]))
            cen_s[...] = dodd
            cc = cen_s[:, pl.ds(1, 28, 2), :]
            center = jnp.pad(cc, ((1, 0), (1, 0), (0, 0)))

            out_ref[0, :, :, sl] = jnp.where(center > t, pmax, pmean)


def kernel(x):
    return pl.pallas_call(
        _fused_kernel,
        grid=(2 * _N,),
        in_specs=[
            pl.BlockSpec((1, _H, _W, _C),
                         lambda i: (jax.lax.rem(i, _N), 0, 0, 0)),
        ],
        out_specs=pl.BlockSpec((1, _OH, _OW, _C),
                               lambda i: (jnp.maximum(i - _N, 0), 0, 0, 0)),
        out_shape=jax.ShapeDtypeStruct((_N, _OH, _OW, _C), jnp.float32),
        scratch_shapes=[
            pltpu.VMEM((60, _PW, _C), jnp.float32),
            pltpu.VMEM((_OH, 60, _CB), jnp.float32),
            pltpu.VMEM((_OH, 60, _CB), jnp.float32),
            pltpu.VMEM((28, 58, _CB), jnp.float32),
            pltpu.VMEM((1, _C), jnp.float32),
            pltpu.SMEM((1,), jnp.float32),
        ],
        compiler_params=pltpu.CompilerParams(
            dimension_semantics=("arbitrary",)),
    )(x)
